# speedup vs baseline: 2.4153x; 2.4153x over previous
import functools

import jax
import jax.numpy as jnp
from jax import lax
from jax.experimental import pallas as pl
from jax.experimental.pallas import tpu as pltpu
from jax.experimental.pallas import tpu_sc as plsc

_D = 128
_NC = 2
_NS = 16
_NW = _NC * _NS
_CH = 128


@functools.partial(jax.jit, static_argnums=0)
def _gather(B, idx2d, tbl):
    n_ch = B // (_NW * _CH)

    def body(idx_hbm, table_hbm, out_hbm, idx_v, rows_v, gsem):
        wid = lax.axis_index("s") * _NC + lax.axis_index("c")
        row0 = wid * n_ch
        pltpu.sync_copy(idx_hbm.at[pl.ds(row0, n_ch), :], idx_v)

        def chunk(g, carry):
            pltpu.async_copy(table_hbm.at[idx_v.at[g]], rows_v, gsem).wait()
            pltpu.sync_copy(rows_v, out_hbm.at[pl.ds((row0 + g) * _CH, _CH), :])
            return carry

        lax.fori_loop(0, n_ch, chunk, 0)

    mesh = plsc.VectorSubcoreMesh(core_axis_name="c", subcore_axis_name="s")
    f = pl.kernel(
        body,
        out_type=jax.ShapeDtypeStruct((B, _D), jnp.float32),
        mesh=mesh,
        scratch_types=[
            pltpu.VMEM((n_ch, _CH), jnp.int32),
            pltpu.VMEM((_CH, _D), jnp.float32),
            pltpu.SemaphoreType.DMA,
        ],
    )
    return f(idx2d, tbl)


def kernel(Z, table):
    n, m = Z.shape
    B = n * m
    idx2d = Z.reshape(B // _CH, _CH).astype(jnp.int32)
    tbl = table.at[0].set(0.0)
    out = _gather(B, idx2d, tbl)
    return out.reshape(n, m, _D)
